# baseline (speedup 1.0000x reference)
import functools

import jax
import jax.numpy as jnp
from jax import lax
from jax.experimental import pallas as pl
from jax.experimental.pallas import tpu as pltpu
from jax.experimental.pallas import tpu_sc as plsc

K = 8192
D = 64
N_TOK = 16384
T = 256
GRID = N_TOK // T
KC = 1024
NKC = K // KC
INV_N = 1.0 / N_TOK
COMMIT = 0.25
DN = (((1,), (1,)), ((), ()))


def _dist_kernel(z_ref, e_ref, idx_ref, loss_ref, ppl_ref, cnt_ref):
    i = pl.program_id(0)
    z = z_ref[...]
    e = e_ref[...]
    z2 = jnp.sum(z * z, axis=1, keepdims=True)
    e2 = jnp.sum(e * e, axis=1)
    dT = lax.dot_general(e, z, DN, preferred_element_type=jnp.float32)
    scores = (z2 - 2.0 * dT.T) + e2[None, :]

    H = K // 2
    col = lax.broadcasted_iota(jnp.int32, (T, H), 1)
    s0 = scores[:, :H]
    s1 = scores[:, H:]
    m0 = jnp.min(s0, axis=1, keepdims=True)
    i0 = jnp.min(jnp.where(s0 == m0, col, K), axis=1)
    m1 = jnp.min(s1, axis=1, keepdims=True)
    i1 = jnp.min(jnp.where(s1 == m1, col, K), axis=1) + H
    m0b = m0.astype(jnp.bfloat16).astype(jnp.float32)
    take1 = m1[:, 0] < m0b[:, 0]
    idx = jnp.where(take1, i1, i0)
    idx_ref[...] = idx[None, None, :]

    tile_sum = jnp.sum(jnp.where(take1, m1[:, 0], m0[:, 0]))

    @pl.when(i == 0)
    def _():
        cnt_ref[...] = jnp.zeros((1, K), jnp.float32)
        loss_ref[...] = jnp.zeros((1, 1), jnp.float32)

    loss_ref[...] = loss_ref[...] + tile_sum
    for c in range(NKC):
        code_ids = lax.broadcasted_iota(jnp.int32, (1, KC), 1) + c * KC
        eq = (idx[:, None] == code_ids).astype(jnp.float32)
        part = jnp.sum(eq, axis=0, keepdims=True)
        cnt_ref[0:1, c * KC:(c + 1) * KC] = cnt_ref[0:1, c * KC:(c + 1) * KC] + part

    @pl.when(i == GRID - 1)
    def _():
        loss_ref[...] = loss_ref[...] * (COMMIT / (N_TOK * D))
        p = cnt_ref[...] * INV_N
        ent = jnp.sum(p * jnp.log(p + 1e-10))
        ppl_ref[...] = jnp.broadcast_to(jnp.exp(-ent), (1, 1))


_dist_call = pl.pallas_call(
    _dist_kernel,
    grid=(GRID,),
    in_specs=[
        pl.BlockSpec((T, D), lambda i: (i, 0)),
        pl.BlockSpec((K, D), lambda i: (0, 0)),
    ],
    out_specs=[
        pl.BlockSpec((1, 1, T), lambda i: (i, 0, 0)),
        pl.BlockSpec((1, 1), lambda i: (0, 0)),
        pl.BlockSpec((1, 1), lambda i: (0, 0)),
    ],
    out_shape=[
        jax.ShapeDtypeStruct((GRID, 1, T), jnp.int32),
        jax.ShapeDtypeStruct((1, 1), jnp.float32),
        jax.ShapeDtypeStruct((1, 1), jnp.float32),
    ],
    scratch_shapes=[pltpu.VMEM((1, K), jnp.float32)],
)


_NC = 2
_NS = 16
NW = _NC * _NS
BPW = N_TOK // NW
CH = 128
NCH = BPW // CH
DP = 128


def _gather_body(emb_hbm, idx_hbm, out_hbm, idx_v, rows_v, sem):
    wid = lax.axis_index("s") * _NC + lax.axis_index("c")
    pltpu.sync_copy(idx_hbm.at[wid], idx_v)
    copies = [
        pltpu.async_copy(emb_hbm.at[idx_v.at[j]],
                         rows_v.at[pl.ds(j * CH, CH)], sem)
        for j in range(NCH)
    ]
    for cp in copies:
        cp.wait()
    pltpu.sync_copy(rows_v, out_hbm.at[pl.ds(wid * BPW, BPW)])


@functools.cache
def _sc_gather_fn():
    return pl.kernel(
        _gather_body,
        mesh=plsc.VectorSubcoreMesh(core_axis_name="c", subcore_axis_name="s",
                                    num_cores=_NC),
        out_type=jax.ShapeDtypeStruct((N_TOK, DP), jnp.float32),
        scratch_types=[
            pltpu.VMEM((NCH, CH), jnp.int32),
            pltpu.VMEM((BPW, DP), jnp.float32),
            pltpu.SemaphoreType.DMA,
        ],
    )


def kernel(z_e, embedding):
    z = jnp.transpose(z_e, (0, 2, 3, 1)).reshape(N_TOK, D)
    idx3, loss, ppl = _dist_call(z, embedding)
    idx = idx3.reshape(N_TOK)
    emb_pad = jnp.pad(embedding, ((0, 0), (0, DP - D)))
    qflat = _sc_gather_fn()(emb_pad, idx.reshape(NW, NCH, CH))[:, :D]
    quant = qflat.reshape(16, 32, 32, 64).transpose(0, 3, 1, 2)
    return (quant, loss.reshape(()), ppl.reshape(()),
            idx.reshape(16, 32, 32))
